# baseline (speedup 1.0000x reference)
import functools

import jax
import jax.numpy as jnp
from jax import lax
from jax.experimental import pallas as pl
from jax.experimental.pallas import tpu as pltpu
from jax.experimental.pallas import tpu_sc as plsc

VOCAB = 30522
DIM = 768
MAX_POS = 512
BATCH = 4
SEQ = 512

NW = 32
TOK = BATCH * SEQ
TPW = TOK // NW
BT = 128


def _gather_sc(ids_flat, word_embeddings):
    mesh = plsc.VectorSubcoreMesh(core_axis_name="c", subcore_axis_name="s")

    @functools.partial(
        pl.kernel,
        mesh=mesh,
        out_type=jax.ShapeDtypeStruct((TOK, DIM), jnp.float32),
        scratch_types=[
            pltpu.VMEM((TPW,), jnp.int32),
            pltpu.VMEM((TPW, DIM), jnp.float32),
            pltpu.SemaphoreType.DMA,
            pltpu.SemaphoreType.DMA,
            pltpu.SemaphoreType.DMA,
        ],
    )
    def body(ids_hbm, word_hbm, out_hbm, idx_v, rows_v, isem, gsem, ssem):
        wid = lax.axis_index("s") * 2 + lax.axis_index("c")
        base = wid * TPW
        pltpu.async_copy(ids_hbm.at[pl.ds(base, TPW)], idx_v, isem).wait()
        pltpu.async_copy(word_hbm.at[idx_v], rows_v, gsem).wait()
        pltpu.async_copy(rows_v, out_hbm.at[pl.ds(base, TPW)], ssem).wait()

    return body(ids_flat, word_embeddings)


def _ln_tc_body(g_ref, p_ref, gam_ref, bet_ref, o_ref):
    v = g_ref[...] + p_ref[...]
    m = jnp.mean(v, axis=-1, keepdims=True)
    c = v - m
    var = jnp.mean(c * c, axis=-1, keepdims=True)
    o_ref[...] = (c * lax.rsqrt(var + 1e-12)) * gam_ref[...] + bet_ref[...]


def _ln_tc(gathered, position_embeddings, gamma, beta):
    return pl.pallas_call(
        _ln_tc_body,
        grid=(TOK // BT,),
        in_specs=[
            pl.BlockSpec((BT, DIM), lambda i: (i, 0)),
            pl.BlockSpec((BT, DIM), lambda i: (i % (SEQ // BT), 0)),
            pl.BlockSpec((1, DIM), lambda i: (0, 0)),
            pl.BlockSpec((1, DIM), lambda i: (0, 0)),
        ],
        out_specs=pl.BlockSpec((BT, DIM), lambda i: (i, 0)),
        out_shape=jax.ShapeDtypeStruct((TOK, DIM), jnp.float32),
    )(gathered, position_embeddings, gamma[None, :], beta[None, :])


def kernel(input_ids, word_embeddings, position_embeddings, gamma, beta):
    ids_flat = input_ids.reshape(TOK).astype(jnp.int32)
    gathered = _gather_sc(ids_flat, word_embeddings)
    out = _ln_tc(word_embeddings[:TOK], position_embeddings, gamma, beta)
    out = out + 0.0 * gathered[:1, :1]
    return out.reshape(BATCH, SEQ, DIM)


# speedup vs baseline: 1.4782x; 1.4782x over previous
import functools

import jax
import jax.numpy as jnp
from jax import lax
from jax.experimental import pallas as pl
from jax.experimental.pallas import tpu as pltpu
from jax.experimental.pallas import tpu_sc as plsc

VOCAB = 30522
DIM = 768
MAX_POS = 512
BATCH = 4
SEQ = 512

NW = 32
TOK = BATCH * SEQ
TPW = TOK // NW
BT = 512


def _gather_sc(ids_flat, word_embeddings):
    mesh = plsc.VectorSubcoreMesh(core_axis_name="c", subcore_axis_name="s")

    @functools.partial(
        pl.kernel,
        mesh=mesh,
        out_type=jax.ShapeDtypeStruct((TOK, DIM), jnp.float32),
        scratch_types=[
            pltpu.VMEM((TPW,), jnp.int32),
            pltpu.VMEM((TPW, DIM), jnp.float32),
            pltpu.SemaphoreType.DMA,
            pltpu.SemaphoreType.DMA,
            pltpu.SemaphoreType.DMA,
        ],
    )
    def body(ids_hbm, word_hbm, out_hbm, idx_v, rows_v, isem, gsem, ssem):
        wid = lax.axis_index("s") * 2 + lax.axis_index("c")
        base = wid * TPW
        h = TPW // 2
        pltpu.async_copy(ids_hbm.at[pl.ds(base, TPW)], idx_v, isem).wait()
        g1 = pltpu.async_copy(word_hbm.at[idx_v.at[pl.ds(0, h)]],
                              rows_v.at[pl.ds(0, h)], gsem)
        g2 = pltpu.async_copy(word_hbm.at[idx_v.at[pl.ds(h, h)]],
                              rows_v.at[pl.ds(h, h)], gsem)
        g1.wait()
        s1 = pltpu.async_copy(rows_v.at[pl.ds(0, h)],
                              out_hbm.at[pl.ds(base, h)], ssem)
        g2.wait()
        s2 = pltpu.async_copy(rows_v.at[pl.ds(h, h)],
                              out_hbm.at[pl.ds(base + h, h)], ssem)
        s1.wait()
        s2.wait()

    return body(ids_flat, word_embeddings)


def _ln_tc_body(g_ref, p_ref, gam_ref, bet_ref, o_ref):
    v = g_ref[...] + p_ref[...]
    m = jnp.mean(v, axis=-1, keepdims=True)
    c = v - m
    var = jnp.mean(c * c, axis=-1, keepdims=True)
    o_ref[...] = (c * lax.rsqrt(var + 1e-12)) * gam_ref[...] + bet_ref[...]


def _ln_tc(gathered, position_embeddings, gamma, beta):
    return pl.pallas_call(
        _ln_tc_body,
        grid=(TOK // BT,),
        in_specs=[
            pl.BlockSpec((BT, DIM), lambda i: (i, 0)),
            pl.BlockSpec((BT, DIM), lambda i: (i % (SEQ // BT), 0)),
            pl.BlockSpec((1, DIM), lambda i: (0, 0)),
            pl.BlockSpec((1, DIM), lambda i: (0, 0)),
        ],
        out_specs=pl.BlockSpec((BT, DIM), lambda i: (i, 0)),
        out_shape=jax.ShapeDtypeStruct((TOK, DIM), jnp.float32),
    )(gathered, position_embeddings, gamma[None, :], beta[None, :])


def kernel(input_ids, word_embeddings, position_embeddings, gamma, beta):
    ids_flat = input_ids.reshape(TOK).astype(jnp.int32)
    gathered = _gather_sc(ids_flat, word_embeddings)
    out = _ln_tc(gathered, position_embeddings, gamma, beta)
    return out.reshape(BATCH, SEQ, DIM)


# speedup vs baseline: 1.5329x; 1.0371x over previous
import functools

import jax
import jax.numpy as jnp
from jax import lax
from jax.experimental import pallas as pl
from jax.experimental.pallas import tpu as pltpu
from jax.experimental.pallas import tpu_sc as plsc

VOCAB = 30522
DIM = 768
MAX_POS = 512
BATCH = 4
SEQ = 512

NW = 32
TOK = BATCH * SEQ
TPW = TOK // NW
BT = 1024


def _gather_sc(ids_flat, word_embeddings):
    mesh = plsc.VectorSubcoreMesh(core_axis_name="c", subcore_axis_name="s")

    @functools.partial(
        pl.kernel,
        mesh=mesh,
        out_type=jax.ShapeDtypeStruct((TOK, DIM), jnp.float32),
        scratch_types=[
            pltpu.VMEM((TPW,), jnp.int32),
            pltpu.VMEM((TPW, DIM), jnp.float32),
            pltpu.SemaphoreType.DMA,
            pltpu.SemaphoreType.DMA,
            pltpu.SemaphoreType.DMA,
        ],
    )
    def body(ids_hbm, word_hbm, out_hbm, idx_v, rows_v, isem, gsem, ssem):
        wid = lax.axis_index("s") * 2 + lax.axis_index("c")
        base = wid * TPW
        h = TPW // 2
        pltpu.async_copy(ids_hbm.at[pl.ds(base, TPW)], idx_v, isem).wait()
        g1 = pltpu.async_copy(word_hbm.at[idx_v.at[pl.ds(0, h)]],
                              rows_v.at[pl.ds(0, h)], gsem)
        g2 = pltpu.async_copy(word_hbm.at[idx_v.at[pl.ds(h, h)]],
                              rows_v.at[pl.ds(h, h)], gsem)
        g1.wait()
        s1 = pltpu.async_copy(rows_v.at[pl.ds(0, h)],
                              out_hbm.at[pl.ds(base, h)], ssem)
        g2.wait()
        s2 = pltpu.async_copy(rows_v.at[pl.ds(h, h)],
                              out_hbm.at[pl.ds(base + h, h)], ssem)
        s1.wait()
        s2.wait()

    return body(ids_flat, word_embeddings)


def _ln_tc_body(g_ref, p_ref, gam_ref, bet_ref, o_ref):
    g = g_ref[...].reshape(BT // SEQ, SEQ, DIM)
    v = (g + p_ref[...][None]).reshape(BT, DIM)
    m = jnp.mean(v, axis=-1, keepdims=True)
    c = v - m
    var = jnp.mean(c * c, axis=-1, keepdims=True)
    o_ref[...] = (c * lax.rsqrt(var + 1e-12)) * gam_ref[...] + bet_ref[...]


def _ln_tc(gathered, position_embeddings, gamma, beta):
    return pl.pallas_call(
        _ln_tc_body,
        grid=(TOK // BT,),
        in_specs=[
            pl.BlockSpec((BT, DIM), lambda i: (i, 0)),
            pl.BlockSpec((SEQ, DIM), lambda i: (0, 0)),
            pl.BlockSpec((1, DIM), lambda i: (0, 0)),
            pl.BlockSpec((1, DIM), lambda i: (0, 0)),
        ],
        out_specs=pl.BlockSpec((BT, DIM), lambda i: (i, 0)),
        out_shape=jax.ShapeDtypeStruct((TOK, DIM), jnp.float32),
    )(gathered, position_embeddings, gamma[None, :], beta[None, :])


def kernel(input_ids, word_embeddings, position_embeddings, gamma, beta):
    ids_flat = input_ids.reshape(TOK).astype(jnp.int32)
    gathered = _gather_sc(ids_flat, word_embeddings)
    out = _ln_tc(gathered, position_embeddings, gamma, beta)
    return out.reshape(BATCH, SEQ, DIM)
